# Initial kernel scaffold; baseline (speedup 1.0000x reference)
import jax
import jax.numpy as jnp
from jax.experimental import pallas as pl


def kernel(inputs):
    raise NotImplementedError("write your pallas kernel here")


# speedup vs baseline: 4.4231x; 4.4231x over previous
import functools

import numpy as np
import jax
import jax.numpy as jnp
from jax.experimental import pallas as pl

_DIAG = 2
_SEQ = 512
_FLAT = _SEQ * _SEQ
_NSLAB = 128
_SG = 8

_r, _c = np.triu_indices(_SEQ, k=_DIAG)
_N = _r.size
_NROW = _SEQ - _DIAG
_OFF = np.concatenate([[0], np.cumsum(_SEQ - _DIAG - np.arange(_NROW))]).astype(int)


def _body(in_ref, out_ref):
    for i in range(_NROW):
        ln = _SEQ - _DIAG - i
        src = i * (_SEQ + 1) + _DIAG
        dst = int(_OFF[i])
        out_ref[:, pl.ds(dst, ln)] = in_ref[:, pl.ds(src, ln)]


def kernel(inputs):
    b, ch, s, _ = inputs.shape
    flat = inputs.reshape(_NSLAB, _FLAT)
    out = pl.pallas_call(
        _body,
        out_shape=jax.ShapeDtypeStruct((_NSLAB, _N), jnp.float32),
        grid=(_NSLAB // _SG,),
        in_specs=[pl.BlockSpec((_SG, _FLAT), lambda g: (g, 0))],
        out_specs=pl.BlockSpec((_SG, _N), lambda g: (g, 0)),
    )(flat)
    return out.reshape(b, ch, _N)
